# Initial kernel scaffold; baseline (speedup 1.0000x reference)
import jax
import jax.numpy as jnp
from jax.experimental import pallas as pl


def kernel(actions, action_tokens):
    raise NotImplementedError("write your pallas kernel here")


# speedup vs baseline: 2.8428x; 2.8428x over previous
import functools

import jax
import jax.numpy as jnp
from jax import lax
from jax.experimental import pallas as pl
from jax.experimental.pallas import tpu as pltpu
from jax.experimental.pallas import tpu_sc as plsc

NUM_ACTIONS = 1000
MODEL_DIM = 128
BATCH = 16384
HIST = 50

NC = 2
NS = 16
NW = NC * NS

B = BATCH * HIST
B_PER_W = B // NW
SUB = 128
NSUB = 4
CHUNK = SUB * NSUB
N_STEPS = B_PER_W // CHUNK
IDX_ROWS_PER_W = B_PER_W // SUB


def _make_gather():
    mesh = plsc.VectorSubcoreMesh(core_axis_name="c", subcore_axis_name="s")

    @functools.partial(
        pl.kernel,
        mesh=mesh,
        out_type=jax.ShapeDtypeStruct((B, MODEL_DIM), jnp.float32),
        scratch_types=[
            pltpu.VMEM((NSUB, SUB), jnp.int32),
            pltpu.VMEM((CHUNK, MODEL_DIM), jnp.float32),
            pltpu.SemaphoreType.DMA,
        ],
    )
    def k(table_hbm, idx_hbm, out_hbm, idx_v, rows_v, sem):
        wid = lax.axis_index("s") * NC + lax.axis_index("c")
        idx_row0 = wid * IDX_ROWS_PER_W
        out0 = wid * B_PER_W

        def step(i, carry):
            pltpu.sync_copy(idx_hbm.at[pl.ds(idx_row0 + i * NSUB, NSUB)], idx_v)
            copies = [
                pltpu.async_copy(
                    table_hbm.at[idx_v.at[j]],
                    rows_v.at[pl.ds(j * SUB, SUB)],
                    sem,
                )
                for j in range(NSUB)
            ]
            for cp in copies:
                cp.wait()
            pltpu.sync_copy(rows_v, out_hbm.at[pl.ds(out0 + i * CHUNK, CHUNK)])
            return carry

        lax.fori_loop(0, N_STEPS, step, 0)

    return k


_gather = _make_gather()


def kernel(actions, action_tokens):
    idx = actions.reshape(B // SUB, SUB).astype(jnp.int32)
    out = _gather(action_tokens, idx)
    return out.astype(jnp.bfloat16).reshape(BATCH, HIST, MODEL_DIM)


# speedup vs baseline: 3.0475x; 1.0720x over previous
import functools

import jax
import jax.numpy as jnp
from jax import lax
from jax.experimental import pallas as pl
from jax.experimental.pallas import tpu as pltpu
from jax.experimental.pallas import tpu_sc as plsc

NUM_ACTIONS = 1000
MODEL_DIM = 128
BATCH = 16384
HIST = 50

NC = 2
NS = 16
NW = NC * NS

B = BATCH * HIST
W_TOTAL = B // 2
W_PER_WORKER = W_TOTAL // NW
SUB = 128
NSUB = 4
CHUNK_W = SUB * NSUB
N_STEPS = W_PER_WORKER // CHUNK_W
IDX_ROWS_PER_W = W_PER_WORKER // SUB
CHUNK_WORDS = CHUNK_W * MODEL_DIM


def _make_gather():
    mesh = plsc.VectorSubcoreMesh(core_axis_name="c", subcore_axis_name="s")

    @functools.partial(
        pl.kernel,
        mesh=mesh,
        out_type=jax.ShapeDtypeStruct((B, MODEL_DIM), jnp.bfloat16),
        scratch_types=[
            pltpu.VMEM((NSUB, SUB), jnp.int32),
            pltpu.VMEM((NSUB, SUB), jnp.int32),
            pltpu.VMEM((CHUNK_W, MODEL_DIM), jnp.int32),
            pltpu.SemaphoreType.DMA,
        ],
    )
    def k(lo_hbm, hi_hbm, ie_hbm, io_hbm, out_hbm, ie_v, io_v, wbuf, sem):
        wid = lax.axis_index("s") * NC + lax.axis_index("c")
        idx_row0 = wid * IDX_ROWS_PER_W
        out_row0 = wid * W_PER_WORKER
        out_words = out_hbm.bitcast(jnp.int32)

        def step(i, carry):
            r0 = idx_row0 + i * NSUB
            pltpu.sync_copy(ie_hbm.at[pl.ds(r0, NSUB)], ie_v)
            pltpu.sync_copy(io_hbm.at[pl.ds(r0, NSUB)], io_v)
            los = [
                pltpu.async_copy(
                    lo_hbm.at[ie_v.at[j]],
                    wbuf.at[pl.ds(j * SUB, SUB)],
                    sem,
                )
                for j in range(NSUB)
            ]
            for cp in los:
                cp.wait()
            his = [
                pltpu.async_copy(
                    hi_hbm.at[io_v.at[j]],
                    wbuf.at[pl.ds(j * SUB, SUB)],
                    sem,
                    add=True,
                )
                for j in range(NSUB)
            ]
            for cp in his:
                cp.wait()
            pltpu.sync_copy(
                wbuf,
                out_words.at[pl.ds(out_row0 + i * CHUNK_W, CHUNK_W)],
            )
            return carry

        lax.fori_loop(0, N_STEPS, step, 0)

    return k


_gather = _make_gather()


def kernel(actions, action_tokens):
    bits = jax.lax.bitcast_convert_type(
        action_tokens.astype(jnp.bfloat16), jnp.uint16
    ).astype(jnp.int32)
    lo_t = bits
    hi_t = bits << 16
    pairs = actions.reshape(W_TOTAL, 2).astype(jnp.int32)
    idx_e = pairs[:, 0].reshape(W_TOTAL // SUB, SUB)
    idx_o = pairs[:, 1].reshape(W_TOTAL // SUB, SUB)
    out = _gather(lo_t, hi_t, idx_e, idx_o)
    return out.reshape(BATCH, HIST, MODEL_DIM)


# speedup vs baseline: 3.1190x; 1.0235x over previous
import functools

import jax
import jax.numpy as jnp
from jax import lax
from jax.experimental import pallas as pl
from jax.experimental.pallas import tpu as pltpu
from jax.experimental.pallas import tpu_sc as plsc

NUM_ACTIONS = 1000
MODEL_DIM = 128
BATCH = 16384
HIST = 50

NC = 2
NS = 16
NW = NC * NS

B = BATCH * HIST
W_TOTAL = B // 2
W_PER_WORKER = W_TOTAL // NW
SUB = 128
NSUB = 2
CHUNK_W = SUB * NSUB
N_STEPS = W_PER_WORKER // CHUNK_W
IDX_ROWS_PER_W = W_PER_WORKER // SUB
IDX_ROWS_PAD = 104


def _make_gather():
    mesh = plsc.VectorSubcoreMesh(core_axis_name="c", subcore_axis_name="s")

    @functools.partial(
        pl.kernel,
        mesh=mesh,
        out_type=jax.ShapeDtypeStruct((B, MODEL_DIM), jnp.bfloat16),
        scratch_types=[
            pltpu.VMEM((IDX_ROWS_PAD, SUB), jnp.int32),
            pltpu.VMEM((IDX_ROWS_PAD, SUB), jnp.int32),
            pltpu.VMEM((CHUNK_W, MODEL_DIM), jnp.int32),
            pltpu.VMEM((CHUNK_W, MODEL_DIM), jnp.int32),
            pltpu.SemaphoreType.DMA,
            pltpu.SemaphoreType.DMA,
            pltpu.SemaphoreType.DMA,
            pltpu.SemaphoreType.DMA,
        ],
    )
    def k(lo_hbm, hi_hbm, ie_hbm, io_hbm, out_hbm,
          ie_v, io_v, wb0, wb1, ls0, ls1, hs0, hs1):
        wid = lax.axis_index("s") * NC + lax.axis_index("c")
        out_row0 = wid * W_PER_WORKER
        out_words = out_hbm.bitcast(jnp.int32)

        wbufs = (wb0, wb1)
        lsems = (ls0, ls1)
        hsems = (hs0, hs1)

        pltpu.sync_copy(ie_hbm.at[wid], ie_v)
        pltpu.sync_copy(io_hbm.at[wid], io_v)

        def fire_lo(step, b):
            for j in range(NSUB):
                pltpu.async_copy(
                    lo_hbm.at[ie_v.at[step * NSUB + j]],
                    wbufs[b].at[pl.ds(j * SUB, SUB)],
                    lsems[b],
                )

        def fire_hi(step, b):
            for j in range(NSUB):
                pltpu.async_copy(
                    hi_hbm.at[io_v.at[step * NSUB + j]],
                    wbufs[b].at[pl.ds(j * SUB, SUB)],
                    hsems[b],
                    add=True,
                )

        def drain(sem, b):
            pltpu.make_async_copy(
                out_words.at[pl.ds(0, CHUNK_W)], wbufs[b], sem
            ).wait()

        fire_lo(0, 0)
        fire_lo(1, 1)
        drain(ls0, 0)
        fire_hi(0, 0)

        def group(g, carry):
            for b in range(2):
                i = g * 2 + b
                nb = 1 - b
                drain(hsems[b], b)
                pltpu.sync_copy(
                    wbufs[b],
                    out_words.at[pl.ds(out_row0 + i * CHUNK_W, CHUNK_W)],
                )
                @pl.when(i < N_STEPS - 2)
                def _():
                    fire_lo(i + 2, b)
                @pl.when(i < N_STEPS - 1)
                def _():
                    drain(lsems[nb], nb)
                    fire_hi(i + 1, nb)
            return carry

        lax.fori_loop(0, N_STEPS // 2, group, 0)

    return k


_gather = _make_gather()


def kernel(actions, action_tokens):
    bits = jax.lax.bitcast_convert_type(
        action_tokens.astype(jnp.bfloat16), jnp.uint16
    ).astype(jnp.int32)
    lo_t = bits
    hi_t = bits << 16
    pairs = actions.reshape(W_TOTAL, 2).astype(jnp.int32)
    pad = ((0, 0), (0, IDX_ROWS_PAD - IDX_ROWS_PER_W), (0, 0))
    idx_e = jnp.pad(pairs[:, 0].reshape(NW, IDX_ROWS_PER_W, SUB), pad)
    idx_o = jnp.pad(pairs[:, 1].reshape(NW, IDX_ROWS_PER_W, SUB), pad)
    out = _gather(lo_t, hi_t, idx_e, idx_o)
    return out.reshape(BATCH, HIST, MODEL_DIM)


# speedup vs baseline: 6.4771x; 2.0767x over previous
import functools

import jax
import jax.numpy as jnp
from jax import lax
from jax.experimental import pallas as pl
from jax.experimental.pallas import tpu as pltpu
from jax.experimental.pallas import tpu_sc as plsc

NUM_ACTIONS = 1000
MODEL_DIM = 128
BATCH = 16384
HIST = 50

NC = 2
NS = 16
NW = NC * NS

WPB = HIST // 2
B_PER_W = BATCH // NW
NB = 8
N_STEPS = B_PER_W // NB


def _make_gather():
    mesh = plsc.VectorSubcoreMesh(core_axis_name="c", subcore_axis_name="s")

    @functools.partial(
        pl.kernel,
        mesh=mesh,
        out_type=jax.ShapeDtypeStruct((BATCH, HIST, MODEL_DIM), jnp.bfloat16),
        scratch_types=[
            pltpu.VMEM((NB, WPB), jnp.int32),
            pltpu.VMEM((NB, WPB), jnp.int32),
            pltpu.VMEM((NB, WPB), jnp.int32),
            pltpu.VMEM((NB, WPB), jnp.int32),
            pltpu.VMEM((NB, WPB, MODEL_DIM), jnp.int32),
            pltpu.VMEM((NB, WPB, MODEL_DIM), jnp.int32),
            pltpu.SemaphoreType.DMA,
            pltpu.SemaphoreType.DMA,
            pltpu.SemaphoreType.DMA,
            pltpu.SemaphoreType.DMA,
        ],
    )
    def k(lo_hbm, hi_hbm, ie_hbm, io_hbm, out_hbm,
          ie0, ie1, io0, io1, wb0, wb1, ls0, ls1, hs0, hs1):
        wid = lax.axis_index("s") * NC + lax.axis_index("c")
        batch0 = wid * B_PER_W
        out_words = out_hbm.bitcast(jnp.int32)

        ies = (ie0, ie1)
        ios = (io0, io1)
        wbufs = (wb0, wb1)
        lsems = (ls0, ls1)
        hsems = (hs0, hs1)

        def fire_lo(step, b):
            pltpu.sync_copy(ie_hbm.at[pl.ds(batch0 + step * NB, NB)], ies[b])
            for j in range(NB):
                pltpu.async_copy(
                    lo_hbm.at[ies[b].at[j]],
                    wbufs[b].at[j],
                    lsems[b],
                )

        def fire_hi(step, b):
            pltpu.sync_copy(io_hbm.at[pl.ds(batch0 + step * NB, NB)], ios[b])
            for j in range(NB):
                pltpu.async_copy(
                    hi_hbm.at[ios[b].at[j]],
                    wbufs[b].at[j],
                    hsems[b],
                    add=True,
                )

        def drain(sem, b):
            pltpu.make_async_copy(
                out_words.at[pl.ds(0, NB)], wbufs[b], sem
            ).wait()

        fire_lo(0, 0)
        fire_lo(1, 1)
        drain(ls0, 0)
        fire_hi(0, 0)

        def group(g, carry):
            for b in range(2):
                i = g * 2 + b
                nb = 1 - b
                drain(hsems[b], b)
                pltpu.sync_copy(
                    wbufs[b],
                    out_words.at[pl.ds(batch0 + i * NB, NB)],
                )
                @pl.when(i < N_STEPS - 2)
                def _():
                    fire_lo(i + 2, b)
                @pl.when(i < N_STEPS - 1)
                def _():
                    drain(lsems[nb], nb)
                    fire_hi(i + 1, nb)
            return carry

        lax.fori_loop(0, N_STEPS // 2, group, 0)

    return k


_gather = _make_gather()


def kernel(actions, action_tokens):
    bits = jax.lax.bitcast_convert_type(
        action_tokens.astype(jnp.bfloat16), jnp.uint16
    ).astype(jnp.int32)
    lo_t = bits
    hi_t = bits << 16
    acts = actions.astype(jnp.int32)
    idx_e = acts[:, 0::2]
    idx_o = acts[:, 1::2]
    return _gather(lo_t, hi_t, idx_e, idx_o)
